# baseline (speedup 1.0000x reference)
import jax
import jax.numpy as jnp
from jax.experimental import pallas as pl
from jax.experimental.pallas import tpu as pltpu

_DIM = 96
_NH = 8
_HD = _DIM // _NH
_NWIN = 7
_NREG = _NWIN * _NWIN
_RH = 8
_SZ = _RH * _RH
_TOPK = 4
_HID = 4 * _DIM
_SCALE = _DIM ** -0.5
_EPS = 1e-5
_H = 56
_W = 56
_NPIX = _H * _W
_SIDE = 5


def _ln(x, w, b):
    mu = jnp.mean(x, axis=-1, keepdims=True)
    xc = x - mu
    var = jnp.mean(xc * xc, axis=-1, keepdims=True)
    return xc * jax.lax.rsqrt(var + _EPS) * w + b


def _sp2rm(t):
    c = t.shape[-1]
    t = t.reshape(_NWIN, _RH, _NWIN, _RH, c).transpose(0, 2, 1, 3, 4)
    return t.reshape(_NPIX, c)


def _rm2sp(t):
    c = t.shape[-1]
    t = t.reshape(_NWIN, _NWIN, _RH, _RH, c).transpose(0, 2, 1, 3, 4)
    return t.reshape(_NPIX, c)


def _qkv_route_body(x_ref, ln1w_ref, ln1b_ref, qkvw_ref, qkvb_ref,
                    lepe_ref, lepeb_ref,
                    q_ref, k_ref, v_ref, lep_ref, idx_ref):
    x = _sp2rm(jnp.transpose(x_ref[0]))
    xn = _ln(x, ln1w_ref[0], ln1b_ref[0])
    qkv = jnp.dot(xn, qkvw_ref[...], preferred_element_type=jnp.float32)
    qkv = qkv + qkvb_ref[0]
    q = qkv[:, :_DIM]
    k = qkv[:, _DIM:2 * _DIM]
    v = qkv[:, 2 * _DIM:]
    q_ref[0] = (q * _SCALE).astype(jnp.bfloat16)
    k_ref[0] = k.astype(jnp.bfloat16)
    v_ref[0] = v.astype(jnp.bfloat16)

    v3 = _rm2sp(v).reshape(_H, _W, _DIM)
    zrow = jnp.zeros((2, _W + 4, _DIM), jnp.float32)
    zcol = jnp.zeros((_H, 2, _DIM), jnp.float32)
    vp = jnp.concatenate(
        [zrow, jnp.concatenate([zcol, v3, zcol], axis=1), zrow], axis=0)
    acc = jnp.zeros((_H, _W, _DIM), jnp.float32)
    for dy in range(_SIDE):
        for dx in range(_SIDE):
            w = lepe_ref[dy * _SIDE + dx, :][None, None, :]
            acc = acc + vp[dy:dy + _H, dx:dx + _W, :] * w
    lep_ref[0] = acc.reshape(_NPIX, _DIM) + lepeb_ref[0]

    q_p = jnp.mean(q.reshape(_NREG, _SZ, _DIM), axis=1)
    k_p = jnp.mean(k.reshape(_NREG, _SZ, _DIM), axis=1)
    a_r = jnp.dot(q_p, k_p.T, preferred_element_type=jnp.float32)

    iota = jax.lax.broadcasted_iota(
        jnp.int32, (_NREG, _NREG), 1).astype(jnp.float32)
    cols = []
    a_cur = a_r
    for _ in range(_TOPK):
        mx = jnp.max(a_cur, axis=-1, keepdims=True)
        hit = a_cur == mx
        idxf = jnp.min(jnp.where(hit, iota, float(_NREG)), axis=-1,
                       keepdims=True)
        cols.append(idxf)
        a_cur = jnp.where(iota == idxf, -1e30, a_cur)
    idx_ref[0] = jnp.concatenate(cols, axis=-1)


_GRP = 7


def _attn_body(idx_ref, q_ref, k_ref, v_ref, o_ref):
    b = pl.program_id(0)
    p = pl.program_id(1)
    group = []
    for g in range(_GRP):
        r = p * _GRP + g
        q = q_ref[0, g * _SZ:(g + 1) * _SZ, :]
        kgs, vgs = [], []
        for j in range(_TOPK):
            i = idx_ref[b, r, j]
            kgs.append(k_ref[0, pl.ds(i * _SZ, _SZ), :])
            vgs.append(v_ref[0, pl.ds(i * _SZ, _SZ), :])
        kg = jnp.concatenate(kgs, axis=0)
        vg = jnp.concatenate(vgs, axis=0)
        logits = []
        for h in range(_NH):
            lo = h * _HD
            logits.append(jax.lax.dot_general(
                q[:, lo:lo + _HD], kg[:, lo:lo + _HD],
                (((1,), (1,)), ((), ())),
                preferred_element_type=jnp.float32))
        efs = [jnp.exp(l) for l in logits]
        es = [e.astype(jnp.bfloat16) for e in efs]
        ss = [jnp.sum(e, axis=-1, keepdims=True) for e in efs]
        outs = []
        for h in range(_NH):
            lo = h * _HD
            av = jnp.dot(es[h], vg[:, lo:lo + _HD],
                         preferred_element_type=jnp.float32)
            outs.append(av * (1.0 / ss[h]))
        group.append(jnp.concatenate(outs, axis=-1))
    blk = jnp.stack(group, axis=0).reshape(_GRP, _RH, _RH, _DIM)
    blk = blk.transpose(1, 0, 2, 3).reshape(_GRP * _SZ, _DIM)
    o_ref[0] = blk


def _tail_body(x_ref, attn_ref, lep_ref, outw_ref, outb_ref,
               ln2w_ref, ln2b_ref, w1_ref, b1_ref, w2_ref, b2_ref, o_ref):
    t = jnp.transpose(attn_ref[0] + lep_ref[0])
    x1 = x_ref[0] + outb_ref[...] + jnp.dot(
        outw_ref[...], t, preferred_element_type=jnp.float32)
    mu = jnp.mean(x1, axis=0, keepdims=True)
    xc = x1 - mu
    var = jnp.mean(xc * xc, axis=0, keepdims=True)
    y = xc * jax.lax.rsqrt(var + _EPS) * ln2w_ref[...] + ln2b_ref[...]
    hid = jnp.dot(w1_ref[...], y, preferred_element_type=jnp.float32)
    hid = hid + b1_ref[...]
    hid = 0.5 * hid * (1.0 + jax.lax.erf(hid * (2.0 ** -0.5)))
    y2 = jnp.dot(w2_ref[...], hid, preferred_element_type=jnp.float32)
    o_ref[0] = x1 + y2 + b2_ref[...]


def _row2(a):
    return a.reshape(1, -1)


def kernel(x, ln1_w, ln1_b, qkv_w, qkv_b, lepe_w, lepe_b, out_w, out_b,
           ln2_w, ln2_b, mlp_w1, mlp_b1, mlp_w2, mlp_b2):
    n = x.shape[0]
    x_cm = x.reshape(n, _DIM, _NPIX)

    cm_spec = pl.BlockSpec((1, _DIM, _NPIX), lambda b: (b, 0, 0))
    tok_spec = pl.BlockSpec((1, _NPIX, _DIM), lambda b: (b, 0, 0))
    full = lambda s: pl.BlockSpec(s, lambda b: (0,) * len(s))
    lepe25 = lepe_w.reshape(_DIM, _SIDE * _SIDE).T

    q, k, v, lep, idxf = pl.pallas_call(
        _qkv_route_body,
        grid=(n,),
        in_specs=[
            cm_spec,
            full((1, _DIM)), full((1, _DIM)),
            full((_DIM, 3 * _DIM)), full((1, 3 * _DIM)),
            full((_SIDE * _SIDE, _DIM)), full((1, _DIM)),
        ],
        out_specs=[tok_spec, tok_spec, tok_spec, tok_spec,
                   pl.BlockSpec((1, _NREG, _TOPK), lambda b: (b, 0, 0))],
        out_shape=[
            jax.ShapeDtypeStruct((n, _NPIX, _DIM), jnp.bfloat16),
            jax.ShapeDtypeStruct((n, _NPIX, _DIM), jnp.bfloat16),
            jax.ShapeDtypeStruct((n, _NPIX, _DIM), jnp.bfloat16),
            jax.ShapeDtypeStruct((n, _NPIX, _DIM), jnp.float32),
            jax.ShapeDtypeStruct((n, _NREG, _TOPK), jnp.float32),
        ],
        compiler_params=pltpu.CompilerParams(
            dimension_semantics=("parallel",)),
    )(x_cm, _row2(ln1_w), _row2(ln1_b), qkv_w.T, _row2(qkv_b),
      lepe25, _row2(lepe_b))

    idx = idxf.astype(jnp.int32)

    grp_spec = pl.BlockSpec(
        (1, _GRP * _SZ, _DIM), lambda b, p, i_ref: (b, p, 0))
    res_spec = pl.BlockSpec((1, _NPIX, _DIM), lambda b, p, i_ref: (b, 0, 0))

    attn = pl.pallas_call(
        _attn_body,
        grid_spec=pltpu.PrefetchScalarGridSpec(
            num_scalar_prefetch=1,
            grid=(n, _NREG // _GRP),
            in_specs=[grp_spec, res_spec, res_spec],
            out_specs=grp_spec,
        ),
        out_shape=jax.ShapeDtypeStruct((n, _NPIX, _DIM), jnp.float32),
        compiler_params=pltpu.CompilerParams(
            dimension_semantics=("parallel", "arbitrary")),
    )(idx, q, k, v)

    def _col2(a):
        return a.reshape(-1, 1)

    out = pl.pallas_call(
        _tail_body,
        grid=(n,),
        in_specs=[
            cm_spec, tok_spec, tok_spec,
            full((_DIM, _DIM)), full((_DIM, 1)),
            full((_DIM, 1)), full((_DIM, 1)),
            full((_HID, _DIM)), full((_HID, 1)),
            full((_DIM, _HID)), full((_DIM, 1)),
        ],
        out_specs=cm_spec,
        out_shape=jax.ShapeDtypeStruct((n, _DIM, _NPIX), jnp.float32),
        compiler_params=pltpu.CompilerParams(
            dimension_semantics=("parallel",)),
    )(x_cm, attn, lep,
      out_w, _col2(out_b), _col2(ln2_w), _col2(ln2_b),
      mlp_w1, _col2(mlp_b1), mlp_w2, _col2(mlp_b2))

    return out.reshape(n, _DIM, _H, _W)


# speedup vs baseline: 1.0025x; 1.0025x over previous
import jax
import jax.numpy as jnp
from jax.experimental import pallas as pl
from jax.experimental.pallas import tpu as pltpu

_DIM = 96
_NH = 8
_HD = _DIM // _NH
_NWIN = 7
_NREG = _NWIN * _NWIN
_RH = 8
_SZ = _RH * _RH
_TOPK = 4
_HID = 4 * _DIM
_SCALE = _DIM ** -0.5
_EPS = 1e-5
_H = 56
_W = 56
_NPIX = _H * _W
_SIDE = 5


def _ln(x, w, b):
    mu = jnp.mean(x, axis=-1, keepdims=True)
    xc = x - mu
    var = jnp.mean(xc * xc, axis=-1, keepdims=True)
    return xc * jax.lax.rsqrt(var + _EPS) * w + b


def _sp2rm(t):
    c = t.shape[-1]
    t = t.reshape(_NWIN, _RH, _NWIN, _RH, c).transpose(0, 2, 1, 3, 4)
    return t.reshape(_NPIX, c)


def _rm2sp(t):
    c = t.shape[-1]
    t = t.reshape(_NWIN, _NWIN, _RH, _RH, c).transpose(0, 2, 1, 3, 4)
    return t.reshape(_NPIX, c)


def _qkv_route_body(x_ref, ln1w_ref, ln1b_ref, qkvw_ref, qkvb_ref,
                    lepe_ref, lepeb_ref,
                    q_ref, k_ref, v_ref, lep_ref, idx_ref):
    x = _sp2rm(jnp.transpose(x_ref[0]))
    xn = _ln(x, ln1w_ref[0], ln1b_ref[0])
    qkv = jnp.dot(xn, qkvw_ref[...], preferred_element_type=jnp.float32)
    qkv = qkv + qkvb_ref[0]
    q = qkv[:, :_DIM]
    k = qkv[:, _DIM:2 * _DIM]
    v = qkv[:, 2 * _DIM:]
    q_ref[0] = (q * _SCALE).astype(jnp.bfloat16)
    k_ref[0] = k.astype(jnp.bfloat16)
    v_ref[0] = v.astype(jnp.bfloat16)

    v3 = _rm2sp(v).reshape(_H, _W, _DIM)
    zrow = jnp.zeros((2, _W + 4, _DIM), jnp.float32)
    zcol = jnp.zeros((_H, 2, _DIM), jnp.float32)
    vp = jnp.concatenate(
        [zrow, jnp.concatenate([zcol, v3, zcol], axis=1), zrow], axis=0)
    acc = jnp.zeros((_H, _W, _DIM), jnp.float32)
    for dy in range(_SIDE):
        for dx in range(_SIDE):
            w = lepe_ref[dy * _SIDE + dx, :][None, None, :]
            acc = acc + vp[dy:dy + _H, dx:dx + _W, :] * w
    lep_ref[0] = acc.reshape(_NPIX, _DIM) + lepeb_ref[0]

    q_p = jnp.mean(q.reshape(_NREG, _SZ, _DIM), axis=1)
    k_p = jnp.mean(k.reshape(_NREG, _SZ, _DIM), axis=1)
    a_r = jnp.dot(q_p, k_p.T, preferred_element_type=jnp.float32)

    iota = jax.lax.broadcasted_iota(
        jnp.int32, (_NREG, _NREG), 1).astype(jnp.float32)
    cols = []
    a_cur = a_r
    for _ in range(_TOPK):
        mx = jnp.max(a_cur, axis=-1, keepdims=True)
        hit = a_cur == mx
        idxf = jnp.min(jnp.where(hit, iota, float(_NREG)), axis=-1,
                       keepdims=True)
        cols.append(idxf)
        a_cur = jnp.where(iota == idxf, -1e30, a_cur)
    idx_ref[0] = jnp.concatenate(cols, axis=-1).astype(jnp.int32)


_GRP = 7


def _attn_body(idx_ref, q_ref, k_ref, v_ref, o_ref):
    b = pl.program_id(0)
    p = pl.program_id(1)
    group = []
    for g in range(_GRP):
        r = p * _GRP + g
        q = q_ref[0, g * _SZ:(g + 1) * _SZ, :]
        kgs, vgs = [], []
        for j in range(_TOPK):
            i = idx_ref[b, r, j]
            kgs.append(k_ref[0, pl.ds(i * _SZ, _SZ), :])
            vgs.append(v_ref[0, pl.ds(i * _SZ, _SZ), :])
        kg = jnp.concatenate(kgs, axis=0)
        vg = jnp.concatenate(vgs, axis=0)
        logits = []
        for h in range(_NH):
            lo = h * _HD
            logits.append(jax.lax.dot_general(
                q[:, lo:lo + _HD], kg[:, lo:lo + _HD],
                (((1,), (1,)), ((), ())),
                preferred_element_type=jnp.float32))
        efs = [jnp.exp(l) for l in logits]
        es = [e.astype(jnp.bfloat16) for e in efs]
        ss = [jnp.sum(e, axis=-1, keepdims=True) for e in efs]
        outs = []
        for h in range(_NH):
            lo = h * _HD
            av = jnp.dot(es[h], vg[:, lo:lo + _HD],
                         preferred_element_type=jnp.float32)
            outs.append(av * (1.0 / ss[h]))
        group.append(jnp.concatenate(outs, axis=-1))
    blk = jnp.stack(group, axis=0).reshape(_GRP, _RH, _RH, _DIM)
    blk = blk.transpose(1, 0, 2, 3).reshape(_GRP * _SZ, _DIM)
    o_ref[0] = blk


def _tail_body(x_ref, attn_ref, lep_ref, outw_ref, outb_ref,
               ln2w_ref, ln2b_ref, w1_ref, b1_ref, w2_ref, b2_ref, o_ref):
    t = jnp.transpose(attn_ref[0] + lep_ref[0])
    x1 = x_ref[0] + outb_ref[...] + jnp.dot(
        outw_ref[...], t, preferred_element_type=jnp.float32)
    mu = jnp.mean(x1, axis=0, keepdims=True)
    xc = x1 - mu
    var = jnp.mean(xc * xc, axis=0, keepdims=True)
    y = xc * jax.lax.rsqrt(var + _EPS) * ln2w_ref[...] + ln2b_ref[...]
    hid = jnp.dot(w1_ref[...], y, preferred_element_type=jnp.float32)
    hid = hid + b1_ref[...]
    hid = 0.5 * hid * (1.0 + jax.lax.erf(hid * (2.0 ** -0.5)))
    y2 = jnp.dot(w2_ref[...], hid, preferred_element_type=jnp.float32)
    o_ref[0] = x1 + y2 + b2_ref[...]


def _row2(a):
    return a.reshape(1, -1)


def kernel(x, ln1_w, ln1_b, qkv_w, qkv_b, lepe_w, lepe_b, out_w, out_b,
           ln2_w, ln2_b, mlp_w1, mlp_b1, mlp_w2, mlp_b2):
    n = x.shape[0]
    x_cm = x.reshape(n, _DIM, _NPIX)

    cm_spec = pl.BlockSpec((1, _DIM, _NPIX), lambda b: (b, 0, 0))
    tok_spec = pl.BlockSpec((1, _NPIX, _DIM), lambda b: (b, 0, 0))
    full = lambda s: pl.BlockSpec(s, lambda b: (0,) * len(s))
    lepe25 = lepe_w.reshape(_DIM, _SIDE * _SIDE).T

    q, k, v, lep, idx = pl.pallas_call(
        _qkv_route_body,
        grid=(n,),
        in_specs=[
            cm_spec,
            full((1, _DIM)), full((1, _DIM)),
            full((_DIM, 3 * _DIM)), full((1, 3 * _DIM)),
            full((_SIDE * _SIDE, _DIM)), full((1, _DIM)),
        ],
        out_specs=[tok_spec, tok_spec, tok_spec, tok_spec,
                   pl.BlockSpec((1, _NREG, _TOPK), lambda b: (b, 0, 0))],
        out_shape=[
            jax.ShapeDtypeStruct((n, _NPIX, _DIM), jnp.bfloat16),
            jax.ShapeDtypeStruct((n, _NPIX, _DIM), jnp.bfloat16),
            jax.ShapeDtypeStruct((n, _NPIX, _DIM), jnp.bfloat16),
            jax.ShapeDtypeStruct((n, _NPIX, _DIM), jnp.float32),
            jax.ShapeDtypeStruct((n, _NREG, _TOPK), jnp.int32),
        ],
        compiler_params=pltpu.CompilerParams(
            dimension_semantics=("parallel",)),
    )(x_cm, _row2(ln1_w), _row2(ln1_b), qkv_w.T, _row2(qkv_b),
      lepe25, _row2(lepe_b))

    grp_spec = pl.BlockSpec(
        (1, _GRP * _SZ, _DIM), lambda b, p, i_ref: (b, p, 0))
    res_spec = pl.BlockSpec((1, _NPIX, _DIM), lambda b, p, i_ref: (b, 0, 0))

    attn = pl.pallas_call(
        _attn_body,
        grid_spec=pltpu.PrefetchScalarGridSpec(
            num_scalar_prefetch=1,
            grid=(n, _NREG // _GRP),
            in_specs=[grp_spec, res_spec, res_spec],
            out_specs=grp_spec,
        ),
        out_shape=jax.ShapeDtypeStruct((n, _NPIX, _DIM), jnp.float32),
        compiler_params=pltpu.CompilerParams(
            dimension_semantics=("parallel", "arbitrary")),
    )(idx, q, k, v)

    def _col2(a):
        return a.reshape(-1, 1)

    out = pl.pallas_call(
        _tail_body,
        grid=(n,),
        in_specs=[
            cm_spec, tok_spec, tok_spec,
            full((_DIM, _DIM)), full((_DIM, 1)),
            full((_DIM, 1)), full((_DIM, 1)),
            full((_HID, _DIM)), full((_HID, 1)),
            full((_DIM, _HID)), full((_DIM, 1)),
        ],
        out_specs=cm_spec,
        out_shape=jax.ShapeDtypeStruct((n, _DIM, _NPIX), jnp.float32),
        compiler_params=pltpu.CompilerParams(
            dimension_semantics=("parallel",)),
    )(x_cm, attn, lep,
      out_w, _col2(out_b), _col2(ln2_w), _col2(ln2_b),
      mlp_w1, _col2(mlp_b1), mlp_w2, _col2(mlp_b2))

    return out.reshape(n, _DIM, _H, _W)


# speedup vs baseline: 1.0189x; 1.0164x over previous
import jax
import jax.numpy as jnp
from jax.experimental import pallas as pl
from jax.experimental.pallas import tpu as pltpu

_DIM = 96
_NH = 8
_HD = _DIM // _NH
_NWIN = 7
_NREG = _NWIN * _NWIN
_RH = 8
_SZ = _RH * _RH
_TOPK = 4
_HID = 4 * _DIM
_SCALE = _DIM ** -0.5
_EPS = 1e-5
_H = 56
_W = 56
_NPIX = _H * _W
_SIDE = 5


def _ln(x, w, b):
    mu = jnp.mean(x, axis=-1, keepdims=True)
    xc = x - mu
    var = jnp.mean(xc * xc, axis=-1, keepdims=True)
    return xc * jax.lax.rsqrt(var + _EPS) * w + b


def _sp2rm(t):
    c = t.shape[-1]
    t = t.reshape(_NWIN, _RH, _NWIN, _RH, c).transpose(0, 2, 1, 3, 4)
    return t.reshape(_NPIX, c)


def _rm2sp(t):
    c = t.shape[-1]
    t = t.reshape(_NWIN, _NWIN, _RH, _RH, c).transpose(0, 2, 1, 3, 4)
    return t.reshape(_NPIX, c)


def _qkv_route_body(x_ref, ln1w_ref, ln1b_ref, qkvw_ref, qkvb_ref,
                    q_ref, k_ref, v_ref, vf_ref, idx_ref):
    x = _sp2rm(x_ref[0])
    xn = _ln(x, ln1w_ref[0], ln1b_ref[0])
    qkv = jnp.dot(xn, qkvw_ref[...], preferred_element_type=jnp.float32)
    qkv = qkv + qkvb_ref[0]
    q = qkv[:, :_DIM]
    k = qkv[:, _DIM:2 * _DIM]
    v = qkv[:, 2 * _DIM:]
    q_ref[0] = (q * _SCALE).astype(jnp.bfloat16)
    k_ref[0] = k.astype(jnp.bfloat16)
    v_ref[0] = v.astype(jnp.bfloat16)
    vf_ref[0] = _rm2sp(v)

    q_p = jnp.mean(q.reshape(_NREG, _SZ, _DIM), axis=1)
    k_p = jnp.mean(k.reshape(_NREG, _SZ, _DIM), axis=1)
    a_r = jnp.dot(q_p, k_p.T, preferred_element_type=jnp.float32)

    iota = jax.lax.broadcasted_iota(
        jnp.int32, (_NREG, _NREG), 1).astype(jnp.float32)
    cols = []
    a_cur = a_r
    for _ in range(_TOPK):
        mx = jnp.max(a_cur, axis=-1, keepdims=True)
        hit = a_cur == mx
        idxf = jnp.min(jnp.where(hit, iota, float(_NREG)), axis=-1,
                       keepdims=True)
        cols.append(idxf)
        a_cur = jnp.where(iota == idxf, -1e30, a_cur)
    idx_ref[0] = jnp.concatenate(cols, axis=-1).astype(jnp.int32)


_GRP = 7


def _attn_body(idx_ref, q_ref, k_ref, v_ref, o_ref):
    b = pl.program_id(0)
    p = pl.program_id(1)
    group = []
    for g in range(_GRP):
        r = p * _GRP + g
        q = q_ref[0, g * _SZ:(g + 1) * _SZ, :]
        kgs, vgs = [], []
        for j in range(_TOPK):
            i = idx_ref[b, r, j]
            kgs.append(k_ref[0, pl.ds(i * _SZ, _SZ), :])
            vgs.append(v_ref[0, pl.ds(i * _SZ, _SZ), :])
        kg = jnp.concatenate(kgs, axis=0)
        vg = jnp.concatenate(vgs, axis=0)
        logits = []
        for h in range(_NH):
            lo = h * _HD
            logits.append(jax.lax.dot_general(
                q[:, lo:lo + _HD], kg[:, lo:lo + _HD],
                (((1,), (1,)), ((), ())),
                preferred_element_type=jnp.float32))
        efs = [jnp.exp(l) for l in logits]
        es = [e.astype(jnp.bfloat16) for e in efs]
        ss = [jnp.sum(e, axis=-1, keepdims=True) for e in efs]
        outs = []
        for h in range(_NH):
            lo = h * _HD
            av = jnp.dot(es[h], vg[:, lo:lo + _HD],
                         preferred_element_type=jnp.float32)
            outs.append(av * (1.0 / ss[h]))
        group.append(jnp.concatenate(outs, axis=-1))
    blk = jnp.stack(group, axis=0).reshape(_GRP, _RH, _RH, _DIM)
    blk = blk.transpose(1, 0, 2, 3).reshape(_GRP * _SZ, _DIM)
    o_ref[0] = blk


def _tail_body(x_ref, attn_ref, vpad_ref, lepe_ref, lepeb_ref, outw_ref,
               outb_ref, ln2w_ref, ln2b_ref, w1_ref, b1_ref, w2_ref, b2_ref,
               o_ref):
    vpad = vpad_ref[0]
    acc = jnp.zeros((_H, _W, _DIM), jnp.float32)
    for dy in range(_SIDE):
        for dx in range(_SIDE):
            w = lepe_ref[dy * _SIDE + dx, :][None, None, :]
            acc = acc + vpad[dy:dy + _H, dx:dx + _W, :] * w
    lepe = acc.reshape(_NPIX, _DIM) + lepeb_ref[0]

    x = x_ref[0]
    t = jnp.dot(attn_ref[0] + lepe, outw_ref[...],
                preferred_element_type=jnp.float32) + outb_ref[0]
    x1 = x + t
    y = _ln(x1, ln2w_ref[0], ln2b_ref[0])
    hid = jnp.dot(y, w1_ref[...], preferred_element_type=jnp.float32)
    hid = hid + b1_ref[0]
    hid = 0.5 * hid * (1.0 + jax.lax.erf(hid * (2.0 ** -0.5)))
    y2 = jnp.dot(hid, w2_ref[...], preferred_element_type=jnp.float32)
    y2 = y2 + b2_ref[0]
    o_ref[0] = x1 + y2


def _row2(a):
    return a.reshape(1, -1)


def kernel(x, ln1_w, ln1_b, qkv_w, qkv_b, lepe_w, lepe_b, out_w, out_b,
           ln2_w, ln2_b, mlp_w1, mlp_b1, mlp_w2, mlp_b2):
    n = x.shape[0]
    x_sp = jnp.transpose(x, (0, 2, 3, 1)).reshape(n, _NPIX, _DIM)

    tok_spec = pl.BlockSpec((1, _NPIX, _DIM), lambda b: (b, 0, 0))
    full = lambda s: pl.BlockSpec(s, lambda b: (0,) * len(s))

    q, k, v, vf, idx = pl.pallas_call(
        _qkv_route_body,
        grid=(n,),
        in_specs=[
            tok_spec,
            full((1, _DIM)), full((1, _DIM)),
            full((_DIM, 3 * _DIM)), full((1, 3 * _DIM)),
        ],
        out_specs=[tok_spec, tok_spec, tok_spec, tok_spec,
                   pl.BlockSpec((1, _NREG, _TOPK), lambda b: (b, 0, 0))],
        out_shape=[
            jax.ShapeDtypeStruct((n, _NPIX, _DIM), jnp.bfloat16),
            jax.ShapeDtypeStruct((n, _NPIX, _DIM), jnp.bfloat16),
            jax.ShapeDtypeStruct((n, _NPIX, _DIM), jnp.bfloat16),
            jax.ShapeDtypeStruct((n, _NPIX, _DIM), jnp.float32),
            jax.ShapeDtypeStruct((n, _NREG, _TOPK), jnp.int32),
        ],
        compiler_params=pltpu.CompilerParams(
            dimension_semantics=("parallel",)),
    )(x_sp, _row2(ln1_w), _row2(ln1_b), qkv_w.T, _row2(qkv_b))

    grp_spec = pl.BlockSpec(
        (1, _GRP * _SZ, _DIM), lambda b, p, i_ref: (b, p, 0))
    res_spec = pl.BlockSpec((1, _NPIX, _DIM), lambda b, p, i_ref: (b, 0, 0))

    attn = pl.pallas_call(
        _attn_body,
        grid_spec=pltpu.PrefetchScalarGridSpec(
            num_scalar_prefetch=1,
            grid=(n, _NREG // _GRP),
            in_specs=[grp_spec, res_spec, res_spec],
            out_specs=grp_spec,
        ),
        out_shape=jax.ShapeDtypeStruct((n, _NPIX, _DIM), jnp.float32),
        compiler_params=pltpu.CompilerParams(
            dimension_semantics=("parallel", "arbitrary")),
    )(idx, q, k, v)

    v_pad = jnp.pad(vf.reshape(n, _H, _W, _DIM),
                    ((0, 0), (2, 2), (2, 2), (0, 0)))
    lepe25 = lepe_w.reshape(_DIM, _SIDE * _SIDE).T

    out = pl.pallas_call(
        _tail_body,
        grid=(n,),
        in_specs=[
            tok_spec, tok_spec,
            pl.BlockSpec((1, _H + 4, _W + 4, _DIM), lambda b: (b, 0, 0, 0)),
            full((_SIDE * _SIDE, _DIM)), full((1, _DIM)),
            full((_DIM, _DIM)), full((1, _DIM)),
            full((1, _DIM)), full((1, _DIM)),
            full((_DIM, _HID)), full((1, _HID)),
            full((_HID, _DIM)), full((1, _DIM)),
        ],
        out_specs=tok_spec,
        out_shape=jax.ShapeDtypeStruct((n, _NPIX, _DIM), jnp.float32),
        compiler_params=pltpu.CompilerParams(
            dimension_semantics=("parallel",)),
    )(x_sp, attn, v_pad, lepe25, _row2(lepe_b),
      out_w.T, _row2(out_b), _row2(ln2_w), _row2(ln2_b),
      mlp_w1.T, _row2(mlp_b1), mlp_w2.T, _row2(mlp_b2))

    return out.reshape(n, _H, _W, _DIM).transpose(0, 3, 1, 2)
